# baseline (speedup 1.0000x reference)
import functools

import jax
import jax.numpy as jnp
from jax import lax
from jax.experimental import pallas as pl
from jax.experimental.pallas import tpu as pltpu
from jax.experimental.pallas import tpu_sc as plsc

V = 10000
D = 128
HD = D // 2
NDOC = 10000
ALPHA = 0.7
VP = 10240

NC = 2
NS = 16
B = 128
CB = 8


def _make_sc_spmm(nb):
    rows_per_tile = VP // NS
    mesh = plsc.VectorSubcoreMesh(core_axis_name="c", subcore_axis_name="s")

    @functools.partial(
        pl.kernel,
        out_type=jax.ShapeDtypeStruct((NC, VP, HD), jnp.float32),
        mesh=mesh,
        compiler_params=pltpu.CompilerParams(use_tc_tiling_on_sc=False),
        scratch_types=[
            pltpu.VMEM((CB, B), jnp.int32),
            pltpu.VMEM((CB, B), jnp.int32),
            pltpu.VMEM((CB, B), jnp.float32),
            pltpu.VMEM((B, HD), jnp.float32),
            pltpu.VMEM((B, HD), jnp.float32),
            pltpu.VMEM_SHARED((VP, HD), jnp.float32),
            pltpu.VMEM_SHARED((VP, HD), jnp.float32),
            pltpu.SemaphoreType.DMA,
            pltpu.SemaphoreType.DMA,
        ],
    )
    def spmm(rows_hbm, cols_hbm, vals_hbm, table_hbm, zeros_hbm, out_hbm,
             cols_v, rows_v, vals_v, gbuf0, gbuf1, stab, acc, sem0, sem1):
        c = lax.axis_index("c")
        s = lax.axis_index("s")
        r0 = s * rows_per_tile
        pltpu.sync_copy(zeros_hbm.at[pl.ds(r0, rows_per_tile)],
                        acc.at[pl.ds(r0, rows_per_tile)])
        pltpu.sync_copy(table_hbm.at[pl.ds(c * VP + r0, rows_per_tile)],
                        stab.at[pl.ds(r0, rows_per_tile)])
        plsc.subcore_barrier()
        e0 = s * nb
        gbufs = (gbuf0, gbuf1)
        sems = (sem0, sem1)
        nchunks = nb // CB

        def wait_gather(cur):
            pltpu.make_async_copy(stab.at[pl.ds(0, B)], gbufs[cur],
                                  sems[cur]).wait()

        pltpu.sync_copy(cols_hbm.at[pl.ds(e0, CB)], cols_v)
        pltpu.sync_copy(rows_hbm.at[pl.ds(e0, CB)], rows_v)
        pltpu.sync_copy(vals_hbm.at[pl.ds(e0, CB)], vals_v)
        pltpu.async_copy(stab.at[cols_v.at[0]], gbufs[0], sems[0])

        def chunk_body(ci, carry):
            for b in range(CB):
                cur = b % 2
                nxt = 1 - cur
                wait_gather(cur)
                if b + 1 < CB:
                    pltpu.async_copy(stab.at[cols_v.at[b + 1]],
                                     gbufs[nxt], sems[nxt])

                def scale16(k, carry3, _b=b, _cur=cur):
                    gb = gbufs[_cur]
                    vv = vals_v[_b, pl.ds(k * 16, 16)]
                    for t in range(16):
                        i = k * 16 + t
                        v = vv[t]
                        for j in range(HD // 16):
                            sl = pl.ds(j * 16, 16)
                            gb[i, sl] = gb[i, sl] * v
                    return carry3

                lax.fori_loop(0, B // 16, scale16, 0)
                pltpu.sync_copy(gbufs[cur], acc.at[rows_v.at[b]], add=True)

            @pl.when(ci + 1 < nchunks)
            def _():
                base = e0 + (ci + 1) * CB
                pltpu.sync_copy(cols_hbm.at[pl.ds(base, CB)], cols_v)
                pltpu.sync_copy(rows_hbm.at[pl.ds(base, CB)], rows_v)
                pltpu.sync_copy(vals_hbm.at[pl.ds(base, CB)], vals_v)
                pltpu.async_copy(stab.at[cols_v.at[0]], gbufs[0], sems[0])

            return carry

        lax.fori_loop(0, nchunks, chunk_body, 0)

        plsc.subcore_barrier()
        pltpu.sync_copy(acc.at[pl.ds(r0, rows_per_tile)],
                        out_hbm.at[c, pl.ds(r0, rows_per_tile)])

    return spmm


def _pad_edges(idx, vals, nb):
    tot = NS * nb * B
    e = vals.shape[0]
    rows = jnp.concatenate([idx[0], jnp.zeros((tot - e,), idx.dtype)])
    cols = jnp.concatenate([idx[1], jnp.zeros((tot - e,), idx.dtype)])
    v = jnp.concatenate([vals, jnp.zeros((tot - e,), vals.dtype)])
    return (rows.reshape(-1, B).astype(jnp.int32),
            cols.reshape(-1, B).astype(jnp.int32),
            v.reshape(-1, B))


BV = 1024


def _mm_relu_body(p_ref, w_ref, o_ref):
    h = jnp.concatenate([p_ref[0], p_ref[1]], axis=-1)
    h = jnp.maximum(
        jnp.dot(h, w_ref[...], preferred_element_type=jnp.float32), 0.0)
    o_ref[0] = h[:, :HD]
    o_ref[1] = h[:, HD:]


def _stage2_body(p_ref, w_ref, e_ref, g_ref, b_ref, o_ref):
    h = jnp.concatenate([p_ref[0], p_ref[1]], axis=-1)
    h = jnp.maximum(
        jnp.dot(h, w_ref[...], preferred_element_type=jnp.float32), 0.0)
    e = e_ref[...]
    h = (1.0 - ALPHA) * e + ALPHA * h
    mu = jnp.mean(h, axis=1, keepdims=True)
    dlt = h - mu
    var = jnp.mean(dlt * dlt, axis=1, keepdims=True)
    y = dlt * lax.rsqrt(var + 1e-5) * g_ref[...] + b_ref[...] + e
    o_ref[0] = y[:, :HD]
    o_ref[1] = y[:, HD:]


def _stage3_body(q_ref, mw_ref, mb_ref, cw_ref, cb_ref, o_ref):
    h = jnp.concatenate([q_ref[0], q_ref[1]], axis=-1)
    t = jnp.maximum(
        jnp.dot(h, mw_ref[...],
                preferred_element_type=jnp.float32) + mb_ref[...], 0.0)
    o_ref[...] = jnp.dot(t, cw_ref[...],
                         preferred_element_type=jnp.float32) + cb_ref[...]


def kernel(A_indices, A_values, X_indices, X_values, emb, W1, W2, ln_g, ln_b,
           mlp_W, mlp_b, cls_W, cls_b):
    nb_a = (-(-A_values.shape[0] // (NS * B)) + 7) // 8 * 8
    nb_x = (-(-X_values.shape[0] // (NS * B)) + 7) // 8 * 8
    a_rows, a_cols, a_vals = _pad_edges(A_indices, A_values, nb_a)
    x_rows, x_cols, x_vals = _pad_edges(X_indices, X_values, nb_x)
    zeros = jnp.zeros((VP, HD), jnp.float32)
    emb_p = jnp.concatenate([emb, jnp.zeros((VP - V, D), jnp.float32)])
    emb_s = jnp.concatenate([emb_p[:, :HD], emb_p[:, HD:]])

    spmm_a = _make_sc_spmm(nb_a)
    spmm_x = _make_sc_spmm(nb_x)

    grid = VP // BV
    wspec = pl.BlockSpec((D, D), lambda i: (0, 0))
    rowspec = pl.BlockSpec((BV, D), lambda i: (i, 0))
    pspec = pl.BlockSpec((2, BV, HD), lambda i: (0, i, 0))
    vecspec = pl.BlockSpec((1, D), lambda i: (0, 0))

    p1 = spmm_a(a_rows, a_cols, a_vals, emb_s, zeros)
    h1 = pl.pallas_call(
        _mm_relu_body, grid=(grid,),
        in_specs=[pspec, wspec], out_specs=pspec,
        out_shape=jax.ShapeDtypeStruct((2, VP, HD), jnp.float32),
    )(p1, W1)

    p2 = spmm_a(a_rows, a_cols, a_vals, h1.reshape(2 * VP, HD), zeros)
    y = pl.pallas_call(
        _stage2_body, grid=(grid,),
        in_specs=[pspec, wspec, rowspec, vecspec, vecspec], out_specs=pspec,
        out_shape=jax.ShapeDtypeStruct((2, VP, HD), jnp.float32),
    )(p2, W2, emb_p, ln_g.reshape(1, D), ln_b.reshape(1, D))

    q = spmm_x(x_rows, x_cols, x_vals, y.reshape(2 * VP, HD), zeros)

    cls_W_pad = jnp.zeros((D, D), jnp.float32).at[:, :2].set(cls_W)
    cls_b_pad = jnp.zeros((1, D), jnp.float32).at[0, :2].set(cls_b)
    out = pl.pallas_call(
        _stage3_body, grid=(grid,),
        in_specs=[pspec, wspec, vecspec, wspec, vecspec], out_specs=rowspec,
        out_shape=jax.ShapeDtypeStruct((VP, D), jnp.float32),
    )(q, mlp_W, mlp_b.reshape(1, D), cls_W_pad, cls_b_pad)
    return out[:NDOC, :2]


# speedup vs baseline: 1.0262x; 1.0262x over previous
import functools

import jax
import jax.numpy as jnp
from jax import lax
from jax.experimental import pallas as pl
from jax.experimental.pallas import tpu as pltpu
from jax.experimental.pallas import tpu_sc as plsc

V = 10000
D = 128
HD = D // 2
NDOC = 10000
ALPHA = 0.7
VP = 10240

NC = 2
NS = 16
B = 128
CB = 8


def _make_sc_spmm(nb):
    rows_per_tile = VP // NS
    mesh = plsc.VectorSubcoreMesh(core_axis_name="c", subcore_axis_name="s")

    @functools.partial(
        pl.kernel,
        out_type=jax.ShapeDtypeStruct((NC, VP, HD), jnp.float32),
        mesh=mesh,
        compiler_params=pltpu.CompilerParams(use_tc_tiling_on_sc=False),
        scratch_types=[
            pltpu.VMEM((CB, B), jnp.int32),
            pltpu.VMEM((CB, B), jnp.int32),
            pltpu.VMEM((CB, B), jnp.float32),
            pltpu.VMEM((B, HD), jnp.float32),
            pltpu.VMEM((B, HD), jnp.float32),
            pltpu.VMEM_SHARED((VP, HD), jnp.float32),
            pltpu.VMEM_SHARED((VP, HD), jnp.float32),
            pltpu.SemaphoreType.DMA,
            pltpu.SemaphoreType.DMA,
            pltpu.SemaphoreType.DMA,
            pltpu.SemaphoreType.DMA,
        ],
    )
    def spmm(rows_hbm, cols_hbm, vals_hbm, table_hbm, zeros_hbm, out_hbm,
             cols_v, rows_v, vals_v, gbuf0, gbuf1, stab, acc, sem0, sem1,
             ssem0, ssem1):
        c = lax.axis_index("c")
        s = lax.axis_index("s")
        r0 = s * rows_per_tile
        pltpu.sync_copy(zeros_hbm.at[pl.ds(r0, rows_per_tile)],
                        acc.at[pl.ds(r0, rows_per_tile)])
        pltpu.sync_copy(table_hbm.at[pl.ds(c * VP + r0, rows_per_tile)],
                        stab.at[pl.ds(r0, rows_per_tile)])
        plsc.subcore_barrier()
        e0 = s * nb
        gbufs = (gbuf0, gbuf1)
        sems = (sem0, sem1)
        ssems = (ssem0, ssem1)
        nchunks = nb // CB

        def wait_gather(cur):
            pltpu.make_async_copy(stab.at[pl.ds(0, B)], gbufs[cur],
                                  sems[cur]).wait()

        def wait_scatter(cur):
            pltpu.make_async_copy(gbufs[cur], acc.at[rows_v.at[0]],
                                  ssems[cur]).wait()

        pltpu.sync_copy(cols_hbm.at[pl.ds(e0, CB)], cols_v)
        pltpu.sync_copy(rows_hbm.at[pl.ds(e0, CB)], rows_v)
        pltpu.sync_copy(vals_hbm.at[pl.ds(e0, CB)], vals_v)
        pltpu.async_copy(stab.at[cols_v.at[0]], gbufs[0], sems[0])

        def chunk_body(ci, carry):
            for b in range(CB):
                cur = b % 2
                nxt = 1 - cur
                wait_gather(cur)
                if b + 1 < CB:
                    if b >= 1:
                        wait_scatter(nxt)
                    pltpu.async_copy(stab.at[cols_v.at[b + 1]],
                                     gbufs[nxt], sems[nxt])

                def scale16(k, carry3, _b=b, _cur=cur):
                    gb = gbufs[_cur]
                    vv = vals_v[_b, pl.ds(k * 16, 16)]
                    for t in range(16):
                        i = k * 16 + t
                        v = vv[t]
                        for j in range(HD // 16):
                            sl = pl.ds(j * 16, 16)
                            gb[i, sl] = gb[i, sl] * v
                    return carry3

                lax.fori_loop(0, B // 16, scale16, 0)
                pltpu.async_copy(gbufs[cur], acc.at[rows_v.at[b]],
                                 ssems[cur], add=True)

            wait_scatter(0)
            wait_scatter(1)

            @pl.when(ci + 1 < nchunks)
            def _():
                base = e0 + (ci + 1) * CB
                pltpu.sync_copy(cols_hbm.at[pl.ds(base, CB)], cols_v)
                pltpu.sync_copy(rows_hbm.at[pl.ds(base, CB)], rows_v)
                pltpu.sync_copy(vals_hbm.at[pl.ds(base, CB)], vals_v)
                pltpu.async_copy(stab.at[cols_v.at[0]], gbufs[0], sems[0])

            return carry

        lax.fori_loop(0, nchunks, chunk_body, 0)

        plsc.subcore_barrier()
        pltpu.sync_copy(acc.at[pl.ds(r0, rows_per_tile)],
                        out_hbm.at[c, pl.ds(r0, rows_per_tile)])

    return spmm


def _pad_edges(idx, vals, nb):
    tot = NS * nb * B
    e = vals.shape[0]
    rows = jnp.concatenate([idx[0], jnp.zeros((tot - e,), idx.dtype)])
    cols = jnp.concatenate([idx[1], jnp.zeros((tot - e,), idx.dtype)])
    v = jnp.concatenate([vals, jnp.zeros((tot - e,), vals.dtype)])
    return (rows.reshape(-1, B).astype(jnp.int32),
            cols.reshape(-1, B).astype(jnp.int32),
            v.reshape(-1, B))


BV = 1024


def _mm_relu_body(p_ref, w_ref, o_ref):
    h = jnp.concatenate([p_ref[0], p_ref[1]], axis=-1)
    h = jnp.maximum(
        jnp.dot(h, w_ref[...], preferred_element_type=jnp.float32), 0.0)
    o_ref[0] = h[:, :HD]
    o_ref[1] = h[:, HD:]


def _stage2_body(p_ref, w_ref, e_ref, g_ref, b_ref, o_ref):
    h = jnp.concatenate([p_ref[0], p_ref[1]], axis=-1)
    h = jnp.maximum(
        jnp.dot(h, w_ref[...], preferred_element_type=jnp.float32), 0.0)
    e = e_ref[...]
    h = (1.0 - ALPHA) * e + ALPHA * h
    mu = jnp.mean(h, axis=1, keepdims=True)
    dlt = h - mu
    var = jnp.mean(dlt * dlt, axis=1, keepdims=True)
    y = dlt * lax.rsqrt(var + 1e-5) * g_ref[...] + b_ref[...] + e
    o_ref[0] = y[:, :HD]
    o_ref[1] = y[:, HD:]


def _stage3_body(q_ref, mw_ref, mb_ref, cw_ref, cb_ref, o_ref):
    h = jnp.concatenate([q_ref[0], q_ref[1]], axis=-1)
    t = jnp.maximum(
        jnp.dot(h, mw_ref[...],
                preferred_element_type=jnp.float32) + mb_ref[...], 0.0)
    o_ref[...] = jnp.dot(t, cw_ref[...],
                         preferred_element_type=jnp.float32) + cb_ref[...]


def kernel(A_indices, A_values, X_indices, X_values, emb, W1, W2, ln_g, ln_b,
           mlp_W, mlp_b, cls_W, cls_b):
    nb_a = (-(-A_values.shape[0] // (NS * B)) + 7) // 8 * 8
    nb_x = (-(-X_values.shape[0] // (NS * B)) + 7) // 8 * 8
    a_rows, a_cols, a_vals = _pad_edges(A_indices, A_values, nb_a)
    x_rows, x_cols, x_vals = _pad_edges(X_indices, X_values, nb_x)
    zeros = jnp.zeros((VP, HD), jnp.float32)
    emb_p = jnp.concatenate([emb, jnp.zeros((VP - V, D), jnp.float32)])
    emb_s = jnp.concatenate([emb_p[:, :HD], emb_p[:, HD:]])

    spmm_a = _make_sc_spmm(nb_a)
    spmm_x = _make_sc_spmm(nb_x)

    grid = VP // BV
    wspec = pl.BlockSpec((D, D), lambda i: (0, 0))
    rowspec = pl.BlockSpec((BV, D), lambda i: (i, 0))
    pspec = pl.BlockSpec((2, BV, HD), lambda i: (0, i, 0))
    vecspec = pl.BlockSpec((1, D), lambda i: (0, 0))

    p1 = spmm_a(a_rows, a_cols, a_vals, emb_s, zeros)
    h1 = pl.pallas_call(
        _mm_relu_body, grid=(grid,),
        in_specs=[pspec, wspec], out_specs=pspec,
        out_shape=jax.ShapeDtypeStruct((2, VP, HD), jnp.float32),
    )(p1, W1)

    p2 = spmm_a(a_rows, a_cols, a_vals, h1.reshape(2 * VP, HD), zeros)
    y = pl.pallas_call(
        _stage2_body, grid=(grid,),
        in_specs=[pspec, wspec, rowspec, vecspec, vecspec], out_specs=pspec,
        out_shape=jax.ShapeDtypeStruct((2, VP, HD), jnp.float32),
    )(p2, W2, emb_p, ln_g.reshape(1, D), ln_b.reshape(1, D))

    q = spmm_x(x_rows, x_cols, x_vals, y.reshape(2 * VP, HD), zeros)

    cls_W_pad = jnp.zeros((D, D), jnp.float32).at[:, :2].set(cls_W)
    cls_b_pad = jnp.zeros((1, D), jnp.float32).at[0, :2].set(cls_b)
    out = pl.pallas_call(
        _stage3_body, grid=(grid,),
        in_specs=[pspec, wspec, vecspec, wspec, vecspec], out_specs=rowspec,
        out_shape=jax.ShapeDtypeStruct((VP, D), jnp.float32),
    )(q, mlp_W, mlp_b.reshape(1, D), cls_W_pad, cls_b_pad)
    return out[:NDOC, :2]


# speedup vs baseline: 1.3397x; 1.3055x over previous
import functools

import jax
import jax.numpy as jnp
from jax import lax
from jax.experimental import pallas as pl
from jax.experimental.pallas import tpu as pltpu
from jax.experimental.pallas import tpu_sc as plsc

V = 10000
D = 128
HD = D // 2
NDOC = 10000
ALPHA = 0.7
VP = 10240

NC = 2
NS = 16
B = 128
CB = 16


def _make_sc_spmm(nb):
    rows_per_tile = VP // NS
    mesh = plsc.VectorSubcoreMesh(core_axis_name="c", subcore_axis_name="s")

    @functools.partial(
        pl.kernel,
        out_type=jax.ShapeDtypeStruct((NC, VP, HD), jnp.float32),
        mesh=mesh,
        compiler_params=pltpu.CompilerParams(use_tc_tiling_on_sc=False),
        scratch_types=[
            pltpu.VMEM((CB, B), jnp.int32),
            pltpu.VMEM((CB, B), jnp.int32),
            pltpu.VMEM((CB, B), jnp.float32),
            pltpu.VMEM((B, HD), jnp.float32),
            pltpu.VMEM((B, HD), jnp.float32),
            pltpu.VMEM((B, HD), jnp.float32),
            pltpu.VMEM((B, HD), jnp.float32),
            pltpu.VMEM_SHARED((VP, HD), jnp.float32),
            pltpu.VMEM_SHARED((VP, HD), jnp.float32),
        ] + [pltpu.SemaphoreType.DMA] * 8,
    )
    def spmm(rows_hbm, cols_hbm, vals_hbm, table_hbm, zeros_hbm, out_hbm,
             cols_v, rows_v, vals_v, gbuf0, gbuf1, gbuf2, gbuf3, stab, acc,
             sem0, sem1, sem2, sem3, ssem0, ssem1, ssem2, ssem3):
        c = lax.axis_index("c")
        s = lax.axis_index("s")
        r0 = s * rows_per_tile
        pltpu.sync_copy(zeros_hbm.at[pl.ds(r0, rows_per_tile)],
                        acc.at[pl.ds(r0, rows_per_tile)])
        pltpu.sync_copy(table_hbm.at[pl.ds(c * VP + r0, rows_per_tile)],
                        stab.at[pl.ds(r0, rows_per_tile)])
        plsc.subcore_barrier()
        e0 = s * nb
        gbufs = (gbuf0, gbuf1, gbuf2, gbuf3)
        sems = (sem0, sem1, sem2, sem3)
        ssems = (ssem0, ssem1, ssem2, ssem3)
        nchunks = nb // CB

        def wait_gather(cur):
            pltpu.make_async_copy(stab.at[pl.ds(0, B)], gbufs[cur],
                                  sems[cur]).wait()

        def wait_scatter(cur):
            pltpu.make_async_copy(gbufs[cur], acc.at[rows_v.at[0]],
                                  ssems[cur]).wait()

        pltpu.sync_copy(cols_hbm.at[pl.ds(e0, CB)], cols_v)
        pltpu.sync_copy(rows_hbm.at[pl.ds(e0, CB)], rows_v)
        pltpu.sync_copy(vals_hbm.at[pl.ds(e0, CB)], vals_v)
        pltpu.async_copy(stab.at[cols_v.at[0]], gbufs[0], sems[0])

        def chunk_body(ci, carry):
            for b in range(CB):
                cur = b % 4
                wait_gather(cur)
                if b + 1 < CB:
                    nxt = (b + 1) % 4
                    if b >= 3:
                        wait_scatter(nxt)
                    pltpu.async_copy(stab.at[cols_v.at[b + 1]],
                                     gbufs[nxt], sems[nxt])

                def scale16(k, carry3, _b=b, _cur=cur):
                    gb = gbufs[_cur]
                    vv = vals_v[_b, pl.ds(k * 16, 16)]
                    for t in range(16):
                        i = k * 16 + t
                        v = vv[t]
                        for j in range(HD // 16):
                            sl = pl.ds(j * 16, 16)
                            gb[i, sl] = gb[i, sl] * v
                    return carry3

                lax.fori_loop(0, B // 16, scale16, 0)
                pltpu.async_copy(gbufs[cur], acc.at[rows_v.at[b]],
                                 ssems[cur], add=True)

            wait_scatter(0)
            wait_scatter(1)
            wait_scatter(2)
            wait_scatter(3)

            @pl.when(ci + 1 < nchunks)
            def _():
                base = e0 + (ci + 1) * CB
                pltpu.sync_copy(cols_hbm.at[pl.ds(base, CB)], cols_v)
                pltpu.sync_copy(rows_hbm.at[pl.ds(base, CB)], rows_v)
                pltpu.sync_copy(vals_hbm.at[pl.ds(base, CB)], vals_v)
                pltpu.async_copy(stab.at[cols_v.at[0]], gbufs[0], sems[0])

            return carry

        lax.fori_loop(0, nchunks, chunk_body, 0)

        plsc.subcore_barrier()
        pltpu.sync_copy(acc.at[pl.ds(r0, rows_per_tile)],
                        out_hbm.at[c, pl.ds(r0, rows_per_tile)])

    return spmm


def _pad_edges(idx, vals, nb):
    tot = NS * nb * B
    e = vals.shape[0]
    rows = jnp.concatenate([idx[0], jnp.zeros((tot - e,), idx.dtype)])
    cols = jnp.concatenate([idx[1], jnp.zeros((tot - e,), idx.dtype)])
    v = jnp.concatenate([vals, jnp.zeros((tot - e,), vals.dtype)])
    return (rows.reshape(-1, B).astype(jnp.int32),
            cols.reshape(-1, B).astype(jnp.int32),
            v.reshape(-1, B))


BV = 1024


def _mm_relu_body(p_ref, w_ref, o_ref):
    h = jnp.concatenate([p_ref[0], p_ref[1]], axis=-1)
    h = jnp.maximum(
        jnp.dot(h, w_ref[...], preferred_element_type=jnp.float32), 0.0)
    o_ref[0] = h[:, :HD]
    o_ref[1] = h[:, HD:]


def _stage2_body(p_ref, w_ref, e_ref, g_ref, b_ref, o_ref):
    h = jnp.concatenate([p_ref[0], p_ref[1]], axis=-1)
    h = jnp.maximum(
        jnp.dot(h, w_ref[...], preferred_element_type=jnp.float32), 0.0)
    e = e_ref[...]
    h = (1.0 - ALPHA) * e + ALPHA * h
    mu = jnp.mean(h, axis=1, keepdims=True)
    dlt = h - mu
    var = jnp.mean(dlt * dlt, axis=1, keepdims=True)
    y = dlt * lax.rsqrt(var + 1e-5) * g_ref[...] + b_ref[...] + e
    o_ref[0] = y[:, :HD]
    o_ref[1] = y[:, HD:]


def _stage3_body(q_ref, mw_ref, mb_ref, cw_ref, cb_ref, o_ref):
    h = jnp.concatenate([q_ref[0], q_ref[1]], axis=-1)
    t = jnp.maximum(
        jnp.dot(h, mw_ref[...],
                preferred_element_type=jnp.float32) + mb_ref[...], 0.0)
    o_ref[...] = jnp.dot(t, cw_ref[...],
                         preferred_element_type=jnp.float32) + cb_ref[...]


def kernel(A_indices, A_values, X_indices, X_values, emb, W1, W2, ln_g, ln_b,
           mlp_W, mlp_b, cls_W, cls_b):
    nb_a = (-(-A_values.shape[0] // (NS * B)) + CB - 1) // CB * CB
    nb_x = (-(-X_values.shape[0] // (NS * B)) + CB - 1) // CB * CB
    a_rows, a_cols, a_vals = _pad_edges(A_indices, A_values, nb_a)
    x_rows, x_cols, x_vals = _pad_edges(X_indices, X_values, nb_x)
    zeros = jnp.zeros((VP, HD), jnp.float32)
    emb_p = jnp.concatenate([emb, jnp.zeros((VP - V, D), jnp.float32)])
    emb_s = jnp.concatenate([emb_p[:, :HD], emb_p[:, HD:]])

    spmm_a = _make_sc_spmm(nb_a)
    spmm_x = _make_sc_spmm(nb_x)

    grid = VP // BV
    wspec = pl.BlockSpec((D, D), lambda i: (0, 0))
    rowspec = pl.BlockSpec((BV, D), lambda i: (i, 0))
    pspec = pl.BlockSpec((2, BV, HD), lambda i: (0, i, 0))
    vecspec = pl.BlockSpec((1, D), lambda i: (0, 0))

    p1 = spmm_a(a_rows, a_cols, a_vals, emb_s, zeros)
    h1 = pl.pallas_call(
        _mm_relu_body, grid=(grid,),
        in_specs=[pspec, wspec], out_specs=pspec,
        out_shape=jax.ShapeDtypeStruct((2, VP, HD), jnp.float32),
    )(p1, W1)

    p2 = spmm_a(a_rows, a_cols, a_vals, h1.reshape(2 * VP, HD), zeros)
    y = pl.pallas_call(
        _stage2_body, grid=(grid,),
        in_specs=[pspec, wspec, rowspec, vecspec, vecspec], out_specs=pspec,
        out_shape=jax.ShapeDtypeStruct((2, VP, HD), jnp.float32),
    )(p2, W2, emb_p, ln_g.reshape(1, D), ln_b.reshape(1, D))

    q = spmm_x(x_rows, x_cols, x_vals, y.reshape(2 * VP, HD), zeros)

    cls_W_pad = jnp.zeros((D, D), jnp.float32).at[:, :2].set(cls_W)
    cls_b_pad = jnp.zeros((1, D), jnp.float32).at[0, :2].set(cls_b)
    out = pl.pallas_call(
        _stage3_body, grid=(grid,),
        in_specs=[pspec, wspec, vecspec, wspec, vecspec], out_specs=rowspec,
        out_shape=jax.ShapeDtypeStruct((VP, D), jnp.float32),
    )(q, mlp_W, mlp_b.reshape(1, D), cls_W_pad, cls_b_pad)
    return out[:NDOC, :2]
